# Initial kernel scaffold; baseline (speedup 1.0000x reference)
import jax
import jax.numpy as jnp
from jax.experimental import pallas as pl


def kernel(points, depth_img, odometry):
    raise NotImplementedError("write your pallas kernel here")


# speedup vs baseline: 1.4639x; 1.4639x over previous
import functools

import jax
import jax.numpy as jnp
from jax import lax
from jax.experimental import pallas as pl
from jax.experimental.pallas import tpu as pltpu
from jax.experimental.pallas import tpu_sc as plsc

H_IMG = 1024
W_IMG = 2048
N_PTS = 1_000_000
NW = 32
WPER = 31_248
TAIL = N_PTS - NW * WPER
CH = 10_416
NCH = WPER // CH
L = 16


def _tec_body(points_hbm, depth_hbm, coef_hbm, out_hbm,
              u_v, v_v, idx_v, d_v, q0_v, q1_v, coef_v, sem):
    cid = lax.axis_index("c")
    sid = lax.axis_index("s")
    wid = sid * 2 + cid

    pltpu.sync_copy(coef_hbm, coef_v)
    m00 = coef_v[0]
    m01 = coef_v[1]
    m02 = coef_v[2]
    m03 = coef_v[3]
    m10 = coef_v[4]
    m11 = coef_v[5]
    m12 = coef_v[6]
    m13 = coef_v[7]
    m20 = coef_v[8]
    m21 = coef_v[9]
    m22 = coef_v[10]
    m23 = coef_v[11]

    def do_chunk(base, n):
        pltpu.sync_copy(points_hbm.at[pl.ds(base, n)], u_v.at[pl.ds(0, n)])
        pltpu.sync_copy(points_hbm.at[pl.ds(N_PTS + base, n)], v_v.at[pl.ds(0, n)])

        def idx_body(i, c):
            off = i * L
            u = u_v[pl.ds(off, L)]
            v = v_v[pl.ds(off, L)]
            idx_v[pl.ds(off, L)] = (v << 11) + u
            return c

        lax.fori_loop(0, n // L, idx_body, 0)

        pltpu.async_copy(depth_hbm.at[idx_v.at[pl.ds(0, n)]],
                         d_v.at[pl.ds(0, n)], sem).wait()

        def cbody(i, c):
            off = i * L
            pu = u_v[pl.ds(off, L)].astype(jnp.float32)
            pv = v_v[pl.ds(off, L)].astype(jnp.float32)
            d = d_v[pl.ds(off, L)]
            pd0 = pu * d
            pd1 = pv * d
            x = m00 * pd0 + m01 * pd1 + m02 * d + m03
            y = m10 * pd0 + m11 * pd1 + m12 * d + m13
            z = m20 * pd0 + m21 * pd1 + m22 * d + m23
            q0_v[pl.ds(off, L)] = x / z
            q1_v[pl.ds(off, L)] = y / z
            return c

        lax.fori_loop(0, n // L, cbody, 0)

        pltpu.sync_copy(q0_v.at[pl.ds(0, n)], out_hbm.at[pl.ds(base, n)])
        pltpu.sync_copy(q1_v.at[pl.ds(0, n)], out_hbm.at[pl.ds(N_PTS + base, n)])

    for k in range(NCH):
        do_chunk(wid * WPER + k * CH, CH)

    @pl.when(wid == NW - 1)
    def _():
        do_chunk(NW * WPER, TAIL)


@functools.partial(jax.jit, static_argnames=())
def _reproject(points, depth_flat, coef):
    f = functools.partial(
        pl.kernel,
        mesh=plsc.VectorSubcoreMesh(core_axis_name="c", subcore_axis_name="s"),
        out_type=jax.ShapeDtypeStruct((2 * N_PTS,), jnp.float32),
        scratch_types=[
            pltpu.VMEM((CH,), jnp.int32),
            pltpu.VMEM((CH,), jnp.int32),
            pltpu.VMEM((CH,), jnp.int32),
            pltpu.VMEM((CH,), jnp.float32),
            pltpu.VMEM((CH,), jnp.float32),
            pltpu.VMEM((CH,), jnp.float32),
            pltpu.VMEM((12, 16), jnp.float32),
            pltpu.SemaphoreType.DMA,
        ],
    )(_tec_body)
    return f(points, depth_flat, coef)


def kernel(points, depth_img, odometry):
    M = jnp.linalg.inv(odometry)
    coef = jnp.broadcast_to(M[:3, :].reshape(12, 1), (12, 16)).astype(jnp.float32)
    depth_flat = depth_img.reshape(-1)
    out_flat = _reproject(points.reshape(-1), depth_flat, coef)
    return out_flat.reshape(2, N_PTS)


# speedup vs baseline: 1.6596x; 1.1337x over previous
import functools

import jax
import jax.numpy as jnp
from jax import lax
from jax.experimental import pallas as pl
from jax.experimental.pallas import tpu as pltpu
from jax.experimental.pallas import tpu_sc as plsc

H_IMG = 1024
W_IMG = 2048
N_PTS = 1_000_000
NW = 32
WPER = 31_248
TAIL = N_PTS - NW * WPER
CH = 4_464
NCH = WPER // CH
L = 16
UNROLL = 3


def _tec_body(points_hbm, depth_hbm, coef_hbm, out_hbm,
              u_v0, u_v1, v_v0, v_v1, idx_v0, idx_v1, d_v0, d_v1,
              q0_v0, q0_v1, q1_v0, q1_v1, coef_v,
              sem_in0, sem_in1, sem_g0, sem_g1, sem_out0, sem_out1):
    u_v = (u_v0, u_v1)
    v_v = (v_v0, v_v1)
    idx_v = (idx_v0, idx_v1)
    d_v = (d_v0, d_v1)
    q0_v = (q0_v0, q0_v1)
    q1_v = (q1_v0, q1_v1)
    sem_in = (sem_in0, sem_in1)
    sem_g = (sem_g0, sem_g1)
    sem_out = (sem_out0, sem_out1)

    cid = lax.axis_index("c")
    sid = lax.axis_index("s")
    wid = sid * 2 + cid
    w_base = wid * WPER

    pltpu.sync_copy(coef_hbm, coef_v)
    m00 = coef_v[0]
    m01 = coef_v[1]
    m02 = coef_v[2]
    m03 = coef_v[3]
    m10 = coef_v[4]
    m11 = coef_v[5]
    m12 = coef_v[6]
    m13 = coef_v[7]
    m20 = coef_v[8]
    m21 = coef_v[9]
    m22 = coef_v[10]
    m23 = coef_v[11]

    def copyin_start(k):
        b = k & 1
        base = w_base + k * CH
        d1 = pltpu.async_copy(points_hbm.at[pl.ds(base, CH)], u_v[b], sem_in[b])
        d2 = pltpu.async_copy(points_hbm.at[pl.ds(N_PTS + base, CH)],
                              v_v[b], sem_in[b])
        return (d1, d2)

    def idx_stage(k):
        b = k & 1

        def body(i, c):
            off = i * (L * UNROLL)
            for j in range(UNROLL):
                o = off + j * L
                u = u_v[b][pl.ds(o, L)]
                v = v_v[b][pl.ds(o, L)]
                idx_v[b][pl.ds(o, L)] = (v << 11) + u
            return c

        lax.fori_loop(0, CH // (L * UNROLL), body, 0)

    def gather_start(k):
        b = k & 1
        return pltpu.async_copy(depth_hbm.at[idx_v[b]], d_v[b], sem_g[b])

    def compute(k):
        b = k & 1

        def body(i, c):
            off = i * (L * UNROLL)
            for j in range(UNROLL):
                o = off + j * L
                pu = u_v[b][pl.ds(o, L)].astype(jnp.float32)
                pv = v_v[b][pl.ds(o, L)].astype(jnp.float32)
                d = d_v[b][pl.ds(o, L)]
                pd0 = pu * d
                pd1 = pv * d
                x = m00 * pd0 + m01 * pd1 + m02 * d + m03
                y = m10 * pd0 + m11 * pd1 + m12 * d + m13
                z = m20 * pd0 + m21 * pd1 + m22 * d + m23
                q0_v[b][pl.ds(o, L)] = x / z
                q1_v[b][pl.ds(o, L)] = y / z
            return c

        lax.fori_loop(0, CH // (L * UNROLL), body, 0)

    def copyout_start(k):
        b = k & 1
        base = w_base + k * CH
        d1 = pltpu.async_copy(q0_v[b], out_hbm.at[pl.ds(base, CH)], sem_out[b])
        d2 = pltpu.async_copy(q1_v[b], out_hbm.at[pl.ds(N_PTS + base, CH)],
                              sem_out[b])
        return (d1, d2)

    ins = {}
    gs = {}
    outs = {}
    ins[0] = copyin_start(0)
    for dsc in ins[0]:
        dsc.wait()
    idx_stage(0)
    gs[0] = gather_start(0)
    ins[1] = copyin_start(1)
    for k in range(NCH):
        gs[k].wait()
        if k + 1 < NCH:
            for dsc in ins[k + 1]:
                dsc.wait()
            idx_stage(k + 1)
            gs[k + 1] = gather_start(k + 1)
        if k - 2 >= 0:
            for dsc in outs[k - 2]:
                dsc.wait()
        compute(k)
        outs[k] = copyout_start(k)
        if k + 2 < NCH:
            ins[k + 2] = copyin_start(k + 2)
    for k in (NCH - 2, NCH - 1):
        for dsc in outs[k]:
            dsc.wait()

    @pl.when(wid == NW - 1)
    def _():
        base = NW * WPER
        n = TAIL
        pltpu.sync_copy(points_hbm.at[pl.ds(base, n)], u_v[0].at[pl.ds(0, n)])
        pltpu.sync_copy(points_hbm.at[pl.ds(N_PTS + base, n)],
                        v_v[0].at[pl.ds(0, n)])

        def tbody(i, c):
            o = i * L
            u = u_v[0][pl.ds(o, L)]
            v = v_v[0][pl.ds(o, L)]
            idx_v[0][pl.ds(o, L)] = (v << 11) + u
            return c

        lax.fori_loop(0, n // L, tbody, 0)
        pltpu.async_copy(depth_hbm.at[idx_v[0].at[pl.ds(0, n)]],
                         d_v[0].at[pl.ds(0, n)], sem_g[0]).wait()

        def tbody2(i, c):
            o = i * L
            pu = u_v[0][pl.ds(o, L)].astype(jnp.float32)
            pv = v_v[0][pl.ds(o, L)].astype(jnp.float32)
            d = d_v[0][pl.ds(o, L)]
            pd0 = pu * d
            pd1 = pv * d
            x = m00 * pd0 + m01 * pd1 + m02 * d + m03
            y = m10 * pd0 + m11 * pd1 + m12 * d + m13
            z = m20 * pd0 + m21 * pd1 + m22 * d + m23
            q0_v[0][pl.ds(o, L)] = x / z
            q1_v[0][pl.ds(o, L)] = y / z
            return c

        lax.fori_loop(0, n // L, tbody2, 0)
        pltpu.sync_copy(q0_v[0].at[pl.ds(0, n)], out_hbm.at[pl.ds(base, n)])
        pltpu.sync_copy(q1_v[0].at[pl.ds(0, n)],
                        out_hbm.at[pl.ds(N_PTS + base, n)])


@jax.jit
def _reproject(points, depth_flat, coef):
    f = functools.partial(
        pl.kernel,
        mesh=plsc.VectorSubcoreMesh(core_axis_name="c", subcore_axis_name="s"),
        out_type=jax.ShapeDtypeStruct((2 * N_PTS,), jnp.float32),
        scratch_types=[
            pltpu.VMEM((CH,), jnp.int32),
            pltpu.VMEM((CH,), jnp.int32),
            pltpu.VMEM((CH,), jnp.int32),
            pltpu.VMEM((CH,), jnp.int32),
            pltpu.VMEM((CH,), jnp.int32),
            pltpu.VMEM((CH,), jnp.int32),
            pltpu.VMEM((CH,), jnp.float32),
            pltpu.VMEM((CH,), jnp.float32),
            pltpu.VMEM((CH,), jnp.float32),
            pltpu.VMEM((CH,), jnp.float32),
            pltpu.VMEM((CH,), jnp.float32),
            pltpu.VMEM((CH,), jnp.float32),
            pltpu.VMEM((12, 16), jnp.float32),
            pltpu.SemaphoreType.DMA,
            pltpu.SemaphoreType.DMA,
            pltpu.SemaphoreType.DMA,
            pltpu.SemaphoreType.DMA,
            pltpu.SemaphoreType.DMA,
            pltpu.SemaphoreType.DMA,
        ],
    )(_tec_body)
    return f(points, depth_flat, coef)


def kernel(points, depth_img, odometry):
    M = jnp.linalg.inv(odometry)
    coef = jnp.broadcast_to(M[:3, :].reshape(12, 1), (12, 16)).astype(jnp.float32)
    depth_flat = depth_img.reshape(-1)
    out_flat = _reproject(points.reshape(-1), depth_flat, coef)
    return out_flat.reshape(2, N_PTS)
